# baseline (speedup 1.0000x reference)
import functools
import math
import jax
import jax.numpy as jnp
from jax import lax
from jax.experimental import pallas as pl
from jax.experimental.pallas import tpu as pltpu
from jax.experimental.pallas import tpu_sc as plsc

F = 128
NEG_BIG = -3.0e38
POS_BIG = 3.0e38

NW = 32
NT = 320
NSR = 160
NPAD = NW * NT
CHUNK = 4000
GB = 128
QCAP = CHUNK + 3 * GB + 64
ACC_ROWS = NSR + 1
DUMMY = NSR


def _pre_body(x_ref, wp_ref, bp_ref, a_ref, b_ref):
    x = x_ref[...]
    wp = wp_ref[...]
    a_ref[...] = jnp.dot(x, wp[:F, :], preferred_element_type=jnp.float32)
    b_ref[...] = (
        jnp.dot(x, wp[F:, :], preferred_element_type=jnp.float32) + bp_ref[...]
    )


def _pre_project(x, W_pre, b_pre, blk):
    n = x.shape[0]
    grid = (n // blk,)
    return pl.pallas_call(
        _pre_body,
        grid=grid,
        in_specs=[
            pl.BlockSpec((blk, F), lambda i: (i, 0)),
            pl.BlockSpec((2 * F, F), lambda i: (0, 0)),
            pl.BlockSpec((1, F), lambda i: (0, 0)),
        ],
        out_specs=[
            pl.BlockSpec((blk, F), lambda i: (i, 0)),
            pl.BlockSpec((blk, F), lambda i: (i, 0)),
        ],
        out_shape=[
            jax.ShapeDtypeStruct((n, F), jnp.float32),
            jax.ShapeDtypeStruct((n, F), jnp.float32),
        ],
    )(x, W_pre, b_pre.reshape(1, F))


def _post_body(x_ref, a_ref, ssum_ref, ssq_ref, smax_ref, smin_ref, deg_ref,
               wpost_ref, bpost_ref, wlin_ref, blin_ref, avg_ref, out_ref):
    x = x_ref[...]
    a = a_ref[...]
    ssum = ssum_ref[...]
    ssq = ssq_ref[...]
    deg = deg_ref[...]
    deg_c = jnp.maximum(deg, 1.0)
    inv_d = 1.0 / deg_c
    mean = (deg * a + ssum) * inv_d
    r = ssum * inv_d
    var = ssq * inv_d - r * r
    std = jnp.sqrt(jnp.maximum(var, 0.0) + 1e-5)
    has = deg > 0.0
    mx = jnp.where(has, a + smax_ref[...], 0.0)
    mn = jnp.where(has, a + smin_ref[...], 0.0)

    agg = jnp.concatenate([mean, mn, mx, std], axis=-1)
    avg_log = avg_ref[0, 0]
    lg = jnp.log(deg_c + 1.0)
    amp = lg / avg_log
    att = avg_log / lg

    wpost = wpost_ref[...]
    h = jnp.dot(x, wpost[:F, :], preferred_element_type=jnp.float32)
    h += jnp.dot(agg, wpost[F:5 * F, :], preferred_element_type=jnp.float32)
    h += amp * jnp.dot(agg, wpost[5 * F:9 * F, :],
                       preferred_element_type=jnp.float32)
    h += att * jnp.dot(agg, wpost[9 * F:13 * F, :],
                       preferred_element_type=jnp.float32)
    h += bpost_ref[...]
    out_ref[...] = (
        jnp.dot(h, wlin_ref[...], preferred_element_type=jnp.float32)
        + blin_ref[...]
    )


def _post_project(x, a, ssum, ssq, smax, smin, deg, W_post, b_post, W_lin,
                  b_lin, avg_log, blk):
    n = x.shape[0]
    grid = (n // blk,)
    row = pl.BlockSpec((blk, F), lambda i: (i, 0))
    return pl.pallas_call(
        _post_body,
        grid=grid,
        in_specs=[
            row, row, row, row, row, row,
            pl.BlockSpec((blk, 1), lambda i: (i, 0)),
            pl.BlockSpec((13 * F, F), lambda i: (0, 0)),
            pl.BlockSpec((1, F), lambda i: (0, 0)),
            pl.BlockSpec((F, F), lambda i: (0, 0)),
            pl.BlockSpec((1, F), lambda i: (0, 0)),
            pl.BlockSpec((1, 1), lambda i: (0, 0), memory_space=pltpu.SMEM),
        ],
        out_specs=row,
        out_shape=jax.ShapeDtypeStruct((n, F), jnp.float32),
    )(x, a, ssum, ssq, smax, smin, deg, W_post, b_post.reshape(1, F), W_lin,
      b_lin.reshape(1, F), avg_log.reshape(1, 1))


def _sc_body(dst_hbm, src_hbm, b_hbm, ssum_hbm, ssq_hbm, smax_hbm, smin_hbm,
             deg_hbm, dst_buf, src_buf, q_src, q_dloc, gidx, gdloc, rows,
             acc_sum, acc_sq, acc_max, acc_min, deg_acc, sem, semg):
    e_total = dst_hbm.shape[0]
    n_chunks = e_total // CHUNK
    wid = lax.axis_index("s") * 2 + lax.axis_index("c")
    zeros16 = jnp.zeros((16,), jnp.float32)
    neg16 = jnp.full((16,), NEG_BIG, jnp.float32)
    pos16 = jnp.full((16,), POS_BIG, jnp.float32)
    one0 = jnp.where(lax.iota(jnp.int32, 16) == 0, 1.0, 0.0).astype(
        jnp.float32)
    HGB = GB // 2

    def stage_and_issue(off):
        for i in range(GB // 16):
            gidx[pl.ds(i * 16, 16)] = q_src[pl.ds(off + i * 16, 16)]
            gdloc[pl.ds(i * 16, 16)] = q_dloc[pl.ds(off + i * 16, 16)]
        pltpu.async_copy(b_hbm.at[gidx.at[pl.ds(0, HGB)]],
                         rows.at[pl.ds(0, HGB)], semg)
        pltpu.async_copy(b_hbm.at[gidx.at[pl.ds(HGB, HGB)]],
                         rows.at[pl.ds(HGB, HGB)], semg)

    def wait_and_accumulate():
        def grp_body(g, _):
            @pl.when(g == 0)
            def _wa():
                pltpu.make_async_copy(b_hbm.at[gidx.at[pl.ds(0, HGB)]],
                                      rows.at[pl.ds(0, HGB)], semg).wait()

            @pl.when(g == HGB // 16)
            def _wb():
                pltpu.make_async_copy(b_hbm.at[gidx.at[pl.ds(HGB, HGB)]],
                                      rows.at[pl.ds(HGB, HGB)], semg).wait()

            base = g * 16
            dv = gdloc[pl.ds(base, 16)]
            for t in range(16):
                k = base + t
                dl = dv[t]
                for j in range(F // 16):
                    sl = pl.ds(j * 16, 16)
                    r = rows[k, sl]
                    acc_sum[dl, sl] += r
                    acc_sq[dl, sl] += r * r
                    acc_max[dl, sl] = jnp.maximum(acc_max[dl, sl], r)
                    acc_min[dl, sl] = jnp.minimum(acc_min[dl, sl], r)
                deg_acc[pl.ds(dl, 16)] += one0
            return 0

        lax.fori_loop(0, GB // 16, grp_body, 0)

    def sync_flush(off):
        stage_and_issue(off)
        wait_and_accumulate()

    def sub_body(sub, _):
        lo = wid * NT + sub * NSR

        def init_body(r, _):
            for j in range(F // 16):
                sl = pl.ds(j * 16, 16)
                acc_sum[r, sl] = zeros16
                acc_sq[r, sl] = zeros16
                acc_max[r, sl] = neg16
                acc_min[r, sl] = pos16
            return 0

        lax.fori_loop(0, ACC_ROWS, init_body, 0)
        for i in range(192 // 16):
            deg_acc[pl.ds(i * 16, 16)] = zeros16

        lane = lax.iota(jnp.int32, 16)
        dummy16 = jnp.full((16,), DUMMY, jnp.int32)
        zi16 = jnp.zeros((16,), jnp.int32)

        pltpu.async_copy(dst_hbm.at[pl.ds(0, CHUNK)],
                         dst_buf.at[pl.ds(0, CHUNK)], sem)
        pltpu.async_copy(src_hbm.at[pl.ds(0, CHUNK)],
                         src_buf.at[pl.ds(0, CHUNK)], sem)

        def chunk_body(c, carry):
            qpos, pending = carry
            last = c == n_chunks - 1
            par = (c % 2) * CHUNK
            nxt = jnp.minimum(c + 1, n_chunks - 1)
            npar = ((c + 1) % 2) * CHUNK
            pltpu.make_async_copy(dst_hbm.at[pl.ds(c * CHUNK, CHUNK)],
                                  dst_buf.at[pl.ds(par, CHUNK)], sem).wait()
            pltpu.make_async_copy(src_hbm.at[pl.ds(c * CHUNK, CHUNK)],
                                  src_buf.at[pl.ds(par, CHUNK)], sem).wait()
            pltpu.async_copy(dst_hbm.at[pl.ds(nxt * CHUNK, CHUNK)],
                             dst_buf.at[pl.ds(npar, CHUNK)], sem)
            pltpu.async_copy(src_hbm.at[pl.ds(nxt * CHUNK, CHUNK)],
                             src_buf.at[pl.ds(npar, CHUNK)], sem)

            def scan_body(v, qp):
                sl = pl.ds(par + v * 16, 16)
                rel = dst_buf[sl] - lo
                m = (rel >= 0) & (rel < NSR)
                cum = plsc.cumsum(jnp.where(m, 1, 0))
                pos = jnp.where(m, qp + cum - 1, QCAP - 16 + lane)
                plsc.store_scatter(q_src, [pos], src_buf[sl])
                plsc.store_scatter(q_dloc, [pos], rel)
                return qp + cum[15]

            qpos = lax.fori_loop(0, CHUNK // 16, scan_body, qpos, unroll=2)

            @pl.when(pending == 1)
            def _consume():
                wait_and_accumulate()

            def wcond(off):
                return qpos - off >= 2 * GB

            def wbody(off):
                sync_flush(off)
                return off + GB

            off = lax.while_loop(wcond, wbody, 0)

            @pl.when(last)
            def _pad():
                for i in range(2 * GB // 16):
                    q_src[pl.ds(qpos + i * 16, 16)] = zi16
                    q_dloc[pl.ds(qpos + i * 16, 16)] = dummy16

            qeff = jnp.where(last, qpos + 2 * GB, qpos)
            have = qeff - off >= GB

            @pl.when(have)
            def _issue():
                stage_and_issue(off)

            off2 = jnp.where(have, off + GB, off)
            pend2 = jnp.where(have, 1, 0)

            @pl.when(last)
            def _finish():
                @pl.when(pend2 == 1)
                def _c2():
                    wait_and_accumulate()

                sync_flush(off2)

            for i in range(GB // 16):
                sl = pl.ds(i * 16, 16)
                q_src[sl] = q_src[pl.ds(off2 + i * 16, 16)]
                q_dloc[sl] = q_dloc[pl.ds(off2 + i * 16, 16)]
            return (qpos - off2, jnp.where(last, 0, pend2))

        lax.fori_loop(0, n_chunks, chunk_body, (0, 0))

        lpar = (n_chunks % 2) * CHUNK
        pltpu.make_async_copy(dst_hbm.at[pl.ds(0, CHUNK)],
                              dst_buf.at[pl.ds(lpar, CHUNK)], sem).wait()
        pltpu.make_async_copy(src_hbm.at[pl.ds(0, CHUNK)],
                              src_buf.at[pl.ds(lpar, CHUNK)], sem).wait()

        pltpu.sync_copy(acc_sum.at[pl.ds(0, NSR)], ssum_hbm.at[pl.ds(lo, NSR)])
        pltpu.sync_copy(acc_sq.at[pl.ds(0, NSR)], ssq_hbm.at[pl.ds(lo, NSR)])
        pltpu.sync_copy(acc_max.at[pl.ds(0, NSR)], smax_hbm.at[pl.ds(lo, NSR)])
        pltpu.sync_copy(acc_min.at[pl.ds(0, NSR)], smin_hbm.at[pl.ds(lo, NSR)])
        pltpu.sync_copy(deg_acc.at[pl.ds(0, NSR)], deg_hbm.at[pl.ds(lo, NSR)])
        return 0

    lax.fori_loop(0, 2, sub_body, 0)


def _segment_stats_sc(B, src, dst):
    mesh = plsc.VectorSubcoreMesh(core_axis_name="c", subcore_axis_name="s")
    f32 = jnp.float32
    out_type = [
        jax.ShapeDtypeStruct((NPAD, F), f32),
        jax.ShapeDtypeStruct((NPAD, F), f32),
        jax.ShapeDtypeStruct((NPAD, F), f32),
        jax.ShapeDtypeStruct((NPAD, F), f32),
        jax.ShapeDtypeStruct((NPAD,), f32),
    ]
    scratch = [
        pltpu.VMEM((2 * CHUNK,), jnp.int32),
        pltpu.VMEM((2 * CHUNK,), jnp.int32),
        pltpu.VMEM((QCAP,), jnp.int32),
        pltpu.VMEM((QCAP,), jnp.int32),
        pltpu.VMEM((GB,), jnp.int32),
        pltpu.VMEM((GB,), jnp.int32),
        pltpu.VMEM((GB, F), f32),
        pltpu.VMEM((ACC_ROWS, F), f32),
        pltpu.VMEM((ACC_ROWS, F), f32),
        pltpu.VMEM((ACC_ROWS, F), f32),
        pltpu.VMEM((ACC_ROWS, F), f32),
        pltpu.VMEM((192,), f32),
        pltpu.SemaphoreType.DMA,
        pltpu.SemaphoreType.DMA,
    ]
    fn = pl.kernel(_sc_body, out_type=out_type, mesh=mesh,
                   scratch_types=scratch,
                   compiler_params=pltpu.CompilerParams(
                       needs_layout_passes=False))
    return fn(dst, src, B)


def kernel(x, edge_index, W_pre, b_pre, W_post, b_post, W_lin, b_lin):
    n = x.shape[0]
    src = edge_index[0]
    dst = edge_index[1]
    avg_log = jnp.asarray(math.log(33.0), dtype=jnp.float32)

    a, b = _pre_project(x, W_pre, b_pre, blk=1000)
    ssum, ssq, smax, smin, deg = _segment_stats_sc(b, src, dst)
    out = _post_project(x, a, ssum, ssq, smax, smin, deg[:, None], W_post,
                        b_post, W_lin, b_lin, avg_log, blk=1000)
    return out


# speedup vs baseline: 1.0604x; 1.0604x over previous
import functools
import math
import jax
import jax.numpy as jnp
from jax import lax
from jax.experimental import pallas as pl
from jax.experimental.pallas import tpu as pltpu
from jax.experimental.pallas import tpu_sc as plsc

F = 128
NEG_BIG = -3.0e38
POS_BIG = 3.0e38

NW = 32
NT = 320
NSR = 160
NPAD = NW * NT
CHUNK = 4000
GB = 128
QCAP = CHUNK + 3 * GB + 64
ACC_ROWS = NSR + 1
DUMMY = NSR


def _pre_body(x_ref, wp_ref, bp_ref, a_ref, b_ref):
    x = x_ref[...]
    wp = wp_ref[...]
    a_ref[...] = jnp.dot(x, wp[:F, :], preferred_element_type=jnp.float32)
    b_ref[...] = (
        jnp.dot(x, wp[F:, :], preferred_element_type=jnp.float32) + bp_ref[...]
    )


def _pre_project(x, W_pre, b_pre, blk):
    n = x.shape[0]
    grid = (n // blk,)
    return pl.pallas_call(
        _pre_body,
        grid=grid,
        in_specs=[
            pl.BlockSpec((blk, F), lambda i: (i, 0)),
            pl.BlockSpec((2 * F, F), lambda i: (0, 0)),
            pl.BlockSpec((1, F), lambda i: (0, 0)),
        ],
        out_specs=[
            pl.BlockSpec((blk, F), lambda i: (i, 0)),
            pl.BlockSpec((blk, F), lambda i: (i, 0)),
        ],
        out_shape=[
            jax.ShapeDtypeStruct((n, F), jnp.float32),
            jax.ShapeDtypeStruct((n, F), jnp.float32),
        ],
    )(x, W_pre, b_pre.reshape(1, F))


def _post_body(x_ref, a_ref, ssum_ref, ssq_ref, smax_ref, smin_ref, deg_ref,
               wpost_ref, bpost_ref, wlin_ref, blin_ref, avg_ref, out_ref):
    x = x_ref[...]
    a = a_ref[...]
    ssum = ssum_ref[...]
    ssq = ssq_ref[...]
    deg = deg_ref[...]
    deg_c = jnp.maximum(deg, 1.0)
    inv_d = 1.0 / deg_c
    mean = (deg * a + ssum) * inv_d
    r = ssum * inv_d
    var = ssq * inv_d - r * r
    std = jnp.sqrt(jnp.maximum(var, 0.0) + 1e-5)
    has = deg > 0.0
    mx = jnp.where(has, a + smax_ref[...], 0.0)
    mn = jnp.where(has, a + smin_ref[...], 0.0)

    agg = jnp.concatenate([mean, mn, mx, std], axis=-1)
    avg_log = avg_ref[0, 0]
    lg = jnp.log(deg_c + 1.0)
    amp = lg / avg_log
    att = avg_log / lg

    wpost = wpost_ref[...]
    h = jnp.dot(x, wpost[:F, :], preferred_element_type=jnp.float32)
    h += jnp.dot(agg, wpost[F:5 * F, :], preferred_element_type=jnp.float32)
    h += amp * jnp.dot(agg, wpost[5 * F:9 * F, :],
                       preferred_element_type=jnp.float32)
    h += att * jnp.dot(agg, wpost[9 * F:13 * F, :],
                       preferred_element_type=jnp.float32)
    h += bpost_ref[...]
    out_ref[...] = (
        jnp.dot(h, wlin_ref[...], preferred_element_type=jnp.float32)
        + blin_ref[...]
    )


def _post_project(x, a, ssum, ssq, smax, smin, deg, W_post, b_post, W_lin,
                  b_lin, avg_log, blk):
    n = x.shape[0]
    grid = (n // blk,)
    row = pl.BlockSpec((blk, F), lambda i: (i, 0))
    return pl.pallas_call(
        _post_body,
        grid=grid,
        in_specs=[
            row, row, row, row, row, row,
            pl.BlockSpec((blk, 1), lambda i: (i, 0)),
            pl.BlockSpec((13 * F, F), lambda i: (0, 0)),
            pl.BlockSpec((1, F), lambda i: (0, 0)),
            pl.BlockSpec((F, F), lambda i: (0, 0)),
            pl.BlockSpec((1, F), lambda i: (0, 0)),
            pl.BlockSpec((1, 1), lambda i: (0, 0), memory_space=pltpu.SMEM),
        ],
        out_specs=row,
        out_shape=jax.ShapeDtypeStruct((n, F), jnp.float32),
    )(x, a, ssum, ssq, smax, smin, deg, W_post, b_post.reshape(1, F), W_lin,
      b_lin.reshape(1, F), avg_log.reshape(1, 1))


def _sc_body(dst_hbm, src_hbm, b_hbm, ssum_hbm, ssq_hbm, smax_hbm, smin_hbm,
             deg_hbm, dst_buf, src_buf, q_src, q_dloc, gidx, gdloc, rows,
             acc_sum, acc_sq, acc_max, acc_min, deg_acc, sem, semg):
    e_total = dst_hbm.shape[0]
    n_chunks = e_total // CHUNK
    wid = lax.axis_index("s") * 2 + lax.axis_index("c")
    zeros16 = jnp.zeros((16,), jnp.float32)
    neg16 = jnp.full((16,), NEG_BIG, jnp.float32)
    pos16 = jnp.full((16,), POS_BIG, jnp.float32)
    one0 = jnp.where(lax.iota(jnp.int32, 16) == 0, 1.0, 0.0).astype(
        jnp.float32)
    HGB = GB // 2

    def accumulate(off, half):
        def grp_body(g, _):
            base = half * HGB + g * 16
            dv = q_dloc[pl.ds(off + base, 16)]
            for t in range(16):
                k = base + t
                dl = dv[t]
                for j in range(F // 16):
                    sl = pl.ds(j * 16, 16)
                    r = rows[k, sl]
                    acc_sum[dl, sl] += r
                    acc_sq[dl, sl] += r * r
                    acc_max[dl, sl] = jnp.maximum(acc_max[dl, sl], r)
                    acc_min[dl, sl] = jnp.minimum(acc_min[dl, sl], r)
                deg_acc[pl.ds(dl, 16)] += one0
            return 0

        lax.fori_loop(0, HGB // 16, grp_body, 0)

    def flush(off):
        for i in range(GB // 16):
            gidx[pl.ds(i * 16, 16)] = q_src[pl.ds(off + i * 16, 16)]
        ga = pltpu.async_copy(b_hbm.at[gidx.at[pl.ds(0, HGB)]],
                              rows.at[pl.ds(0, HGB)], semg)
        gb = pltpu.async_copy(b_hbm.at[gidx.at[pl.ds(HGB, HGB)]],
                              rows.at[pl.ds(HGB, HGB)], semg)
        ga.wait()
        accumulate(off, 0)
        gb.wait()
        accumulate(off, 1)

    def sub_body(sub, _):
        lo = wid * NT + sub * NSR

        def init_body(r, _):
            for j in range(F // 16):
                sl = pl.ds(j * 16, 16)
                acc_sum[r, sl] = zeros16
                acc_sq[r, sl] = zeros16
                acc_max[r, sl] = neg16
                acc_min[r, sl] = pos16
            return 0

        lax.fori_loop(0, ACC_ROWS, init_body, 0)
        for i in range(192 // 16):
            deg_acc[pl.ds(i * 16, 16)] = zeros16

        lane = lax.iota(jnp.int32, 16)
        dummy16 = jnp.full((16,), DUMMY, jnp.int32)
        zi16 = jnp.zeros((16,), jnp.int32)

        pltpu.async_copy(dst_hbm.at[pl.ds(0, CHUNK)],
                         dst_buf.at[pl.ds(0, CHUNK)], sem)
        pltpu.async_copy(src_hbm.at[pl.ds(0, CHUNK)],
                         src_buf.at[pl.ds(0, CHUNK)], sem)

        def chunk_body(c, qpos):
            last = c == n_chunks - 1
            par = (c % 2) * CHUNK
            nxt = jnp.minimum(c + 1, n_chunks - 1)
            npar = ((c + 1) % 2) * CHUNK
            pltpu.make_async_copy(dst_hbm.at[pl.ds(c * CHUNK, CHUNK)],
                                  dst_buf.at[pl.ds(par, CHUNK)], sem).wait()
            pltpu.make_async_copy(src_hbm.at[pl.ds(c * CHUNK, CHUNK)],
                                  src_buf.at[pl.ds(par, CHUNK)], sem).wait()
            pltpu.async_copy(dst_hbm.at[pl.ds(nxt * CHUNK, CHUNK)],
                             dst_buf.at[pl.ds(npar, CHUNK)], sem)
            pltpu.async_copy(src_hbm.at[pl.ds(nxt * CHUNK, CHUNK)],
                             src_buf.at[pl.ds(npar, CHUNK)], sem)

            def scan_body(v, qp):
                sl = pl.ds(par + v * 16, 16)
                rel = dst_buf[sl] - lo
                m = (rel >= 0) & (rel < NSR)
                cum = plsc.cumsum(jnp.where(m, 1, 0))
                pos = jnp.where(m, qp + cum - 1, QCAP - 16 + lane)
                plsc.store_scatter(q_src, [pos], src_buf[sl])
                plsc.store_scatter(q_dloc, [pos], rel)
                return qp + cum[15]

            qpos = lax.fori_loop(0, CHUNK // 16, scan_body, qpos, unroll=4)

            @pl.when(last)
            def _pad():
                for i in range(GB // 16):
                    q_src[pl.ds(qpos + i * 16, 16)] = zi16
                    q_dloc[pl.ds(qpos + i * 16, 16)] = dummy16

            qeff = jnp.where(last, qpos + GB, qpos)

            def fl_cond(off):
                return off + GB <= qeff

            def fl_body(off):
                flush(off)
                return off + GB

            off = lax.while_loop(fl_cond, fl_body, 0)

            for i in range(GB // 16):
                sl = pl.ds(i * 16, 16)
                q_src[sl] = q_src[pl.ds(off + i * 16, 16)]
                q_dloc[sl] = q_dloc[pl.ds(off + i * 16, 16)]
            return qpos - off

        lax.fori_loop(0, n_chunks, chunk_body, 0)

        lpar = (n_chunks % 2) * CHUNK
        pltpu.make_async_copy(dst_hbm.at[pl.ds(0, CHUNK)],
                              dst_buf.at[pl.ds(lpar, CHUNK)], sem).wait()
        pltpu.make_async_copy(src_hbm.at[pl.ds(0, CHUNK)],
                              src_buf.at[pl.ds(lpar, CHUNK)], sem).wait()

        pltpu.sync_copy(acc_sum.at[pl.ds(0, NSR)], ssum_hbm.at[pl.ds(lo, NSR)])
        pltpu.sync_copy(acc_sq.at[pl.ds(0, NSR)], ssq_hbm.at[pl.ds(lo, NSR)])
        pltpu.sync_copy(acc_max.at[pl.ds(0, NSR)], smax_hbm.at[pl.ds(lo, NSR)])
        pltpu.sync_copy(acc_min.at[pl.ds(0, NSR)], smin_hbm.at[pl.ds(lo, NSR)])
        pltpu.sync_copy(deg_acc.at[pl.ds(0, NSR)], deg_hbm.at[pl.ds(lo, NSR)])
        return 0

    lax.fori_loop(0, 2, sub_body, 0)


def _segment_stats_sc(B, src, dst):
    mesh = plsc.VectorSubcoreMesh(core_axis_name="c", subcore_axis_name="s")
    f32 = jnp.float32
    out_type = [
        jax.ShapeDtypeStruct((NPAD, F), f32),
        jax.ShapeDtypeStruct((NPAD, F), f32),
        jax.ShapeDtypeStruct((NPAD, F), f32),
        jax.ShapeDtypeStruct((NPAD, F), f32),
        jax.ShapeDtypeStruct((NPAD,), f32),
    ]
    scratch = [
        pltpu.VMEM((2 * CHUNK,), jnp.int32),
        pltpu.VMEM((2 * CHUNK,), jnp.int32),
        pltpu.VMEM((QCAP,), jnp.int32),
        pltpu.VMEM((QCAP,), jnp.int32),
        pltpu.VMEM((GB,), jnp.int32),
        pltpu.VMEM((GB,), jnp.int32),
        pltpu.VMEM((GB, F), f32),
        pltpu.VMEM((ACC_ROWS, F), f32),
        pltpu.VMEM((ACC_ROWS, F), f32),
        pltpu.VMEM((ACC_ROWS, F), f32),
        pltpu.VMEM((ACC_ROWS, F), f32),
        pltpu.VMEM((192,), f32),
        pltpu.SemaphoreType.DMA,
        pltpu.SemaphoreType.DMA,
    ]
    fn = pl.kernel(_sc_body, out_type=out_type, mesh=mesh,
                   scratch_types=scratch,
                   compiler_params=pltpu.CompilerParams(
                       needs_layout_passes=False))
    return fn(dst, src, B)


def kernel(x, edge_index, W_pre, b_pre, W_post, b_post, W_lin, b_lin):
    n = x.shape[0]
    src = edge_index[0]
    dst = edge_index[1]
    avg_log = jnp.asarray(math.log(33.0), dtype=jnp.float32)

    a, b = _pre_project(x, W_pre, b_pre, blk=1000)
    ssum, ssq, smax, smin, deg = _segment_stats_sc(b, src, dst)
    out = _post_project(x, a, ssum, ssq, smax, smin, deg[:, None], W_post,
                        b_post, W_lin, b_lin, avg_log, blk=1000)
    return out


# speedup vs baseline: 1.0621x; 1.0015x over previous
import math
import jax
import jax.numpy as jnp
from jax import lax
from jax.experimental import pallas as pl
from jax.experimental.pallas import tpu as pltpu
from jax.experimental.pallas import tpu_sc as plsc

F = 128
NEG_BIG = -3.0e38
POS_BIG = 3.0e38

NW = 32
NT = 320
NSR = 160
NPAD = NW * NT
CHUNK = 4000
GB = 128
QCAP = CHUNK + 3 * GB + 64
ACC_ROWS = NSR + 1
DUMMY = NSR


def _pre_body(x_ref, wp_ref, bp_ref, a_ref, b_ref):
    x = x_ref[...]
    wp = wp_ref[...]
    a_ref[...] = jnp.dot(x, wp[:F, :], preferred_element_type=jnp.float32)
    b_ref[...] = (
        jnp.dot(x, wp[F:, :], preferred_element_type=jnp.float32) + bp_ref[...]
    )


def _pre_project(x, W_pre, b_pre, blk):
    n = x.shape[0]
    grid = (n // blk,)
    return pl.pallas_call(
        _pre_body,
        grid=grid,
        in_specs=[
            pl.BlockSpec((blk, F), lambda i: (i, 0)),
            pl.BlockSpec((2 * F, F), lambda i: (0, 0)),
            pl.BlockSpec((1, F), lambda i: (0, 0)),
        ],
        out_specs=[
            pl.BlockSpec((blk, F), lambda i: (i, 0)),
            pl.BlockSpec((blk, F), lambda i: (i, 0)),
        ],
        out_shape=[
            jax.ShapeDtypeStruct((n, F), jnp.float32),
            jax.ShapeDtypeStruct((n, F), jnp.float32),
        ],
    )(x, W_pre, b_pre.reshape(1, F))


def _post_body(x_ref, a_ref, ssum_ref, ssq_ref, smax_ref, smin_ref, deg_ref,
               wpost_ref, bpost_ref, wlin_ref, blin_ref, avg_ref, out_ref):
    x = x_ref[...]
    a = a_ref[...]
    ssum = ssum_ref[...]
    ssq = ssq_ref[...]
    deg = deg_ref[...]
    deg_c = jnp.maximum(deg, 1.0)
    inv_d = 1.0 / deg_c
    mean = (deg * a + ssum) * inv_d
    r = ssum * inv_d
    var = ssq * inv_d - r * r
    std = jnp.sqrt(jnp.maximum(var, 0.0) + 1e-5)
    has = deg > 0.0
    mx = jnp.where(has, a + smax_ref[...], 0.0)
    mn = jnp.where(has, a + smin_ref[...], 0.0)

    agg = jnp.concatenate([mean, mn, mx, std], axis=-1)
    avg_log = avg_ref[0, 0]
    lg = jnp.log(deg_c + 1.0)
    amp = lg / avg_log
    att = avg_log / lg

    wpost = wpost_ref[...]
    h = jnp.dot(x, wpost[:F, :], preferred_element_type=jnp.float32)
    h += jnp.dot(agg, wpost[F:5 * F, :], preferred_element_type=jnp.float32)
    h += amp * jnp.dot(agg, wpost[5 * F:9 * F, :],
                       preferred_element_type=jnp.float32)
    h += att * jnp.dot(agg, wpost[9 * F:13 * F, :],
                       preferred_element_type=jnp.float32)
    h += bpost_ref[...]
    out_ref[...] = (
        jnp.dot(h, wlin_ref[...], preferred_element_type=jnp.float32)
        + blin_ref[...]
    )


def _post_project(x, a, ssum, ssq, smax, smin, deg, W_post, b_post, W_lin,
                  b_lin, avg_log, blk):
    n = x.shape[0]
    grid = (n // blk,)
    row = pl.BlockSpec((blk, F), lambda i: (i, 0))
    return pl.pallas_call(
        _post_body,
        grid=grid,
        in_specs=[
            row, row, row, row, row, row,
            pl.BlockSpec((blk, 1), lambda i: (i, 0)),
            pl.BlockSpec((13 * F, F), lambda i: (0, 0)),
            pl.BlockSpec((1, F), lambda i: (0, 0)),
            pl.BlockSpec((F, F), lambda i: (0, 0)),
            pl.BlockSpec((1, F), lambda i: (0, 0)),
            pl.BlockSpec((1, 1), lambda i: (0, 0), memory_space=pltpu.SMEM),
        ],
        out_specs=row,
        out_shape=jax.ShapeDtypeStruct((n, F), jnp.float32),
    )(x, a, ssum, ssq, smax, smin, deg, W_post, b_post.reshape(1, F), W_lin,
      b_lin.reshape(1, F), avg_log.reshape(1, 1))


def _sc_body(dst_hbm, src_hbm, b_hbm, ssum_hbm, ssq_hbm, smax_hbm, smin_hbm,
             deg_hbm, dst_buf, src_buf, q_src, q_dloc, gidx, rows,
             acc_sum, acc_sq, acc_max, acc_min, deg_acc, sem, semg):
    e_total = dst_hbm.shape[0]
    n_chunks = e_total // CHUNK
    wid = lax.axis_index("s") * 2 + lax.axis_index("c")
    zeros16 = jnp.zeros((16,), jnp.float32)
    neg16 = jnp.full((16,), NEG_BIG, jnp.float32)
    pos16 = jnp.full((16,), POS_BIG, jnp.float32)
    one0 = jnp.where(lax.iota(jnp.int32, 16) == 0, 1.0, 0.0).astype(
        jnp.float32)
    HGB = GB // 2

    def accumulate(off, half):
        def grp_body(g, _):
            base = half * HGB + g * 16
            dv = q_dloc[pl.ds(off + base, 16)]
            for t in range(16):
                k = base + t
                dl = dv[t]
                for j in range(F // 16):
                    sl = pl.ds(j * 16, 16)
                    r = rows[k, sl]
                    acc_sum[dl, sl] += r
                    acc_sq[dl, sl] += r * r
                    acc_max[dl, sl] = jnp.maximum(acc_max[dl, sl], r)
                    acc_min[dl, sl] = jnp.minimum(acc_min[dl, sl], r)
                deg_acc[pl.ds(dl, 16)] += one0
            return 0

        lax.fori_loop(0, HGB // 16, grp_body, 0)

    def flush(off):
        for i in range(GB // 16):
            gidx[pl.ds(i * 16, 16)] = q_src[pl.ds(off + i * 16, 16)]
        ga = pltpu.async_copy(b_hbm.at[gidx.at[pl.ds(0, HGB)]],
                              rows.at[pl.ds(0, HGB)], semg)
        gb = pltpu.async_copy(b_hbm.at[gidx.at[pl.ds(HGB, HGB)]],
                              rows.at[pl.ds(HGB, HGB)], semg)
        ga.wait()
        accumulate(off, 0)
        gb.wait()
        accumulate(off, 1)

    def sub_body(sub, _):
        lo = wid * NT + sub * NSR

        def init_body(r, _):
            for j in range(F // 16):
                sl = pl.ds(j * 16, 16)
                acc_sum[r, sl] = zeros16
                acc_sq[r, sl] = zeros16
                acc_max[r, sl] = neg16
                acc_min[r, sl] = pos16
            return 0

        lax.fori_loop(0, ACC_ROWS, init_body, 0)
        for i in range(192 // 16):
            deg_acc[pl.ds(i * 16, 16)] = zeros16

        lane = lax.iota(jnp.int32, 16)
        dummy16 = jnp.full((16,), DUMMY, jnp.int32)
        zi16 = jnp.zeros((16,), jnp.int32)

        pltpu.async_copy(dst_hbm.at[pl.ds(0, CHUNK)],
                         dst_buf.at[pl.ds(0, CHUNK)], sem)
        pltpu.async_copy(src_hbm.at[pl.ds(0, CHUNK)],
                         src_buf.at[pl.ds(0, CHUNK)], sem)

        def chunk_body(c, qpos):
            last = c == n_chunks - 1
            par = (c % 2) * CHUNK
            nxt = jnp.minimum(c + 1, n_chunks - 1)
            npar = ((c + 1) % 2) * CHUNK
            pltpu.make_async_copy(dst_hbm.at[pl.ds(c * CHUNK, CHUNK)],
                                  dst_buf.at[pl.ds(par, CHUNK)], sem).wait()
            pltpu.make_async_copy(src_hbm.at[pl.ds(c * CHUNK, CHUNK)],
                                  src_buf.at[pl.ds(par, CHUNK)], sem).wait()
            pltpu.async_copy(dst_hbm.at[pl.ds(nxt * CHUNK, CHUNK)],
                             dst_buf.at[pl.ds(npar, CHUNK)], sem)
            pltpu.async_copy(src_hbm.at[pl.ds(nxt * CHUNK, CHUNK)],
                             src_buf.at[pl.ds(npar, CHUNK)], sem)

            def scan_body(v, qp):
                sl = pl.ds(par + v * 16, 16)
                rel = dst_buf[sl] - lo
                m = (rel >= 0) & (rel < NSR)
                cum = plsc.cumsum(jnp.where(m, 1, 0))
                pos = jnp.where(m, qp + cum - 1, QCAP - 16 + lane)
                plsc.store_scatter(q_src, [pos], src_buf[sl])
                plsc.store_scatter(q_dloc, [pos], rel)
                return qp + cum[15]

            qpos = lax.fori_loop(0, CHUNK // 16, scan_body, qpos, unroll=2)

            @pl.when(last)
            def _pad():
                for i in range(GB // 16):
                    q_src[pl.ds(qpos + i * 16, 16)] = zi16
                    q_dloc[pl.ds(qpos + i * 16, 16)] = dummy16

            qeff = jnp.where(last, qpos + GB, qpos)

            def fl_cond(off):
                return off + GB <= qeff

            def fl_body(off):
                flush(off)
                return off + GB

            off = lax.while_loop(fl_cond, fl_body, 0)

            for i in range(GB // 16):
                sl = pl.ds(i * 16, 16)
                q_src[sl] = q_src[pl.ds(off + i * 16, 16)]
                q_dloc[sl] = q_dloc[pl.ds(off + i * 16, 16)]
            return qpos - off

        lax.fori_loop(0, n_chunks, chunk_body, 0)

        lpar = (n_chunks % 2) * CHUNK
        pltpu.make_async_copy(dst_hbm.at[pl.ds(0, CHUNK)],
                              dst_buf.at[pl.ds(lpar, CHUNK)], sem).wait()
        pltpu.make_async_copy(src_hbm.at[pl.ds(0, CHUNK)],
                              src_buf.at[pl.ds(lpar, CHUNK)], sem).wait()

        pltpu.sync_copy(acc_sum.at[pl.ds(0, NSR)], ssum_hbm.at[pl.ds(lo, NSR)])
        pltpu.sync_copy(acc_sq.at[pl.ds(0, NSR)], ssq_hbm.at[pl.ds(lo, NSR)])
        pltpu.sync_copy(acc_max.at[pl.ds(0, NSR)], smax_hbm.at[pl.ds(lo, NSR)])
        pltpu.sync_copy(acc_min.at[pl.ds(0, NSR)], smin_hbm.at[pl.ds(lo, NSR)])
        pltpu.sync_copy(deg_acc.at[pl.ds(0, NSR)], deg_hbm.at[pl.ds(lo, NSR)])
        return 0

    lax.fori_loop(0, 2, sub_body, 0)


def _segment_stats_sc(B, src, dst):
    mesh = plsc.VectorSubcoreMesh(core_axis_name="c", subcore_axis_name="s")
    f32 = jnp.float32
    out_type = [
        jax.ShapeDtypeStruct((NPAD, F), f32),
        jax.ShapeDtypeStruct((NPAD, F), f32),
        jax.ShapeDtypeStruct((NPAD, F), f32),
        jax.ShapeDtypeStruct((NPAD, F), f32),
        jax.ShapeDtypeStruct((NPAD,), f32),
    ]
    scratch = [
        pltpu.VMEM((2 * CHUNK,), jnp.int32),
        pltpu.VMEM((2 * CHUNK,), jnp.int32),
        pltpu.VMEM((QCAP,), jnp.int32),
        pltpu.VMEM((QCAP,), jnp.int32),
        pltpu.VMEM((GB,), jnp.int32),
        pltpu.VMEM((GB, F), f32),
        pltpu.VMEM((ACC_ROWS, F), f32),
        pltpu.VMEM((ACC_ROWS, F), f32),
        pltpu.VMEM((ACC_ROWS, F), f32),
        pltpu.VMEM((ACC_ROWS, F), f32),
        pltpu.VMEM((192,), f32),
        pltpu.SemaphoreType.DMA,
        pltpu.SemaphoreType.DMA,
    ]
    fn = pl.kernel(_sc_body, out_type=out_type, mesh=mesh,
                   scratch_types=scratch,
                   compiler_params=pltpu.CompilerParams(
                       needs_layout_passes=False))
    return fn(dst, src, B)


def kernel(x, edge_index, W_pre, b_pre, W_post, b_post, W_lin, b_lin):
    n = x.shape[0]
    src = edge_index[0]
    dst = edge_index[1]
    avg_log = jnp.asarray(math.log(33.0), dtype=jnp.float32)

    a, b = _pre_project(x, W_pre, b_pre, blk=1000)
    ssum, ssq, smax, smin, deg = _segment_stats_sc(b, src, dst)
    out = _post_project(x, a, ssum, ssq, smax, smin, deg[:, None], W_post,
                        b_post, W_lin, b_lin, avg_log, blk=1000)
    return out
